# baseline (speedup 1.0000x reference)
import functools

import jax
import jax.numpy as jnp
from jax import lax
from jax.experimental import pallas as pl
from jax.experimental.pallas import tpu as pltpu
from jax.experimental.pallas import tpu_sc as plsc


def kernel(tensor_list, row_embed, col_embed):
    nt, f, _ = tensor_list.shape
    side = int(f ** 0.5)
    h = w = side
    d = row_embed.shape[1]
    assert col_embed.shape[1] == d

    info = plsc.get_sparse_core_info()
    nc, ns = info.num_cores, info.num_subcores
    nw = nc * ns
    rows = (h * w) // nw
    assert rows == w and h == nw

    mesh = plsc.VectorSubcoreMesh(core_axis_name="c", subcore_axis_name="s")

    @functools.partial(
        pl.kernel,
        out_type=jax.ShapeDtypeStruct((nt, h * w, 2 * d), jnp.float32),
        mesh=mesh,
        scratch_types=[
            pltpu.VMEM((rows, 2 * d), jnp.float32),
            pltpu.SemaphoreType.DMA,
        ],
    )
    def pos_kernel(row_hbm, col_hbm, out_hbm, plane_v, sem):
        wid = lax.axis_index("s") * nc + lax.axis_index("c")
        fills = []
        for r in range(rows):
            fills.append(pltpu.async_copy(
                col_hbm.at[pl.ds(r, 1)],
                plane_v.at[pl.ds(r, 1), pl.ds(0, d)], sem))
            fills.append(pltpu.async_copy(
                row_hbm.at[pl.ds(wid, 1)],
                plane_v.at[pl.ds(r, 1), pl.ds(d, d)], sem))
        for cpy in fills:
            cpy.wait()
        outs = []
        for n in range(nt):
            outs.append(pltpu.async_copy(
                plane_v, out_hbm.at[n, pl.ds(wid * rows, rows), :], sem))
        for cpy in outs:
            cpy.wait()

    return pos_kernel(row_embed, col_embed)
